# baseline (speedup 1.0000x reference)
import functools

import jax
import jax.numpy as jnp
from jax import lax
from jax.experimental import pallas as pl
from jax.experimental.pallas import tpu as pltpu
from jax.experimental.pallas import tpu_sc as plsc

D_MODEL = 128
N_TOKENS = 819200
NUM_CORES = 2
NUM_SUBCORES = 16
NUM_WORKERS = NUM_CORES * NUM_SUBCORES
PER_WORKER = N_TOKENS // NUM_WORKERS
CHUNK = 80
N_CHUNKS = PER_WORKER // CHUNK
N_PASSES = 4
P_CHUNKS = N_CHUNKS // N_PASSES
NBUF = 4
LA = 2
MAX_LEN_PAD = 9088
STAGE_ROWS = MAX_LEN_PAD // NUM_SUBCORES
CHUNK_BYTES = CHUNK * D_MODEL * 4


def _sc_gather_body(table_hbm, idx_hbm, out_hbm, tbl_s, idx_v, rows_v, *sems):
    gsems, osems = sems[:NBUF], sems[NBUF:]
    sid = lax.axis_index("s")
    wid = sid * NUM_CORES + lax.axis_index("c")
    base = wid * PER_WORKER

    pltpu.sync_copy(table_hbm.at[pl.ds(sid * STAGE_ROWS, STAGE_ROWS)],
                    tbl_s.at[pl.ds(sid * STAGE_ROWS, STAGE_ROWS)])
    plsc.subcore_barrier()

    def start_gather(j, b):
        src = table_hbm if b == NBUF - 1 else tbl_s
        pltpu.async_copy(src.at[idx_v.at[j]], rows_v.at[b], gsems[b])

    def start_out(g, b):
        pltpu.async_copy(rows_v.at[b],
                         out_hbm.at[pl.ds(base + g * CHUNK, CHUNK)], osems[b])

    def wait_gather(b):
        pltpu.make_async_copy(table_hbm.at[pl.ds(0, CHUNK)], rows_v.at[b],
                              gsems[b]).wait()

    def wait_out(b):
        pltpu.make_async_copy(rows_v.at[b], out_hbm.at[pl.ds(0, CHUNK)],
                              osems[b]).wait()

    for p in range(N_PASSES):
        pltpu.sync_copy(idx_hbm.at[wid].at[p], idx_v)
        goff = p * P_CHUNKS
        for j in range(LA):
            start_gather(j, j)

        for b in range(NBUF):
            wait_gather(b)
            start_out(goff + b, b)
            bn = (b + LA) % NBUF
            if b + LA >= NBUF:
                wait_out(bn)
            start_gather(b + LA, bn)

        @pl.loop(NBUF, P_CHUNKS - NBUF, step=NBUF)
        def _body(jb):
            for b in range(NBUF):
                j = jb + b
                wait_gather(b)
                start_out(goff + j, b)
                bn = (b + LA) % NBUF
                wait_out(bn)
                start_gather(j + LA, bn)

        for b in range(NBUF):
            j = P_CHUNKS - NBUF + b
            wait_gather(b)
            start_out(goff + j, b)
            if j + LA < P_CHUNKS:
                bn = (b + LA) % NBUF
                wait_out(bn)
                start_gather(j + LA, bn)

        for b in range(NBUF):
            wait_out(b)


@jax.jit
def _sc_gather(x, re):
    idx = x.astype(jnp.int32).reshape(NUM_WORKERS, N_PASSES, P_CHUNKS, CHUNK)
    re_pad = jnp.pad(re, ((0, MAX_LEN_PAD - re.shape[0]), (0, 0)))
    run = pl.kernel(
        _sc_gather_body,
        out_type=jax.ShapeDtypeStruct((N_TOKENS, D_MODEL), jnp.float32),
        mesh=plsc.VectorSubcoreMesh(core_axis_name="c", subcore_axis_name="s"),
        scratch_types=(
            [pltpu.VMEM_SHARED((MAX_LEN_PAD, D_MODEL), jnp.float32),
             pltpu.VMEM((P_CHUNKS, CHUNK), jnp.int32),
             pltpu.VMEM((NBUF, CHUNK, D_MODEL), jnp.float32)]
            + [pltpu.SemaphoreType.DMA] * (2 * NBUF)
        ),
    )
    return run(re_pad, idx)


def kernel(x, re):
    return _sc_gather(x, re)


# speedup vs baseline: 1.1790x; 1.1790x over previous
import functools

import jax
import jax.numpy as jnp
from jax import lax
from jax.experimental import pallas as pl
from jax.experimental.pallas import tpu as pltpu
from jax.experimental.pallas import tpu_sc as plsc

D_MODEL = 128
N_TOKENS = 819200
NUM_CORES = 2
NUM_SUBCORES = 16
NUM_WORKERS = NUM_CORES * NUM_SUBCORES
PER_WORKER = N_TOKENS // NUM_WORKERS
CHUNK = 80
N_CHUNKS = PER_WORKER // CHUNK
N_PASSES = 8
P_CHUNKS = N_CHUNKS // N_PASSES
NBUF = 5
LA = 2
MAX_LEN_PAD = 9088
STAGE_ROWS = MAX_LEN_PAD // NUM_SUBCORES
CHUNK_BYTES = CHUNK * D_MODEL * 4


def _sc_gather_body(table_hbm, idx_hbm, out_hbm, tbl_s, idx_v, rows_v, *sems):
    gsems, osems = sems[:NBUF], sems[NBUF:]
    sid = lax.axis_index("s")
    wid = sid * NUM_CORES + lax.axis_index("c")
    base = wid * PER_WORKER

    pltpu.sync_copy(table_hbm.at[pl.ds(sid * STAGE_ROWS, STAGE_ROWS)],
                    tbl_s.at[pl.ds(sid * STAGE_ROWS, STAGE_ROWS)])
    plsc.subcore_barrier()

    def start_gather(j, b):
        pltpu.async_copy(tbl_s.at[idx_v.at[j]], rows_v.at[b], gsems[b])

    def start_out(g, b):
        pltpu.async_copy(rows_v.at[b],
                         out_hbm.at[pl.ds(base + g * CHUNK, CHUNK)], osems[b])

    def wait_gather(b):
        pltpu.make_async_copy(table_hbm.at[pl.ds(0, CHUNK)], rows_v.at[b],
                              gsems[b]).wait()

    def wait_out(b):
        pltpu.make_async_copy(rows_v.at[b], out_hbm.at[pl.ds(0, CHUNK)],
                              osems[b]).wait()

    for p in range(N_PASSES):
        pltpu.sync_copy(idx_hbm.at[wid].at[p], idx_v)
        goff = p * P_CHUNKS
        for j in range(LA):
            start_gather(j, j)

        for b in range(NBUF):
            wait_gather(b)
            start_out(goff + b, b)
            bn = (b + LA) % NBUF
            if b + LA >= NBUF:
                wait_out(bn)
            start_gather(b + LA, bn)

        @pl.loop(NBUF, P_CHUNKS - NBUF, step=NBUF)
        def _body(jb):
            for b in range(NBUF):
                j = jb + b
                wait_gather(b)
                start_out(goff + j, b)
                bn = (b + LA) % NBUF
                wait_out(bn)
                start_gather(j + LA, bn)

        for b in range(NBUF):
            j = P_CHUNKS - NBUF + b
            wait_gather(b)
            start_out(goff + j, b)
            if j + LA < P_CHUNKS:
                bn = (b + LA) % NBUF
                wait_out(bn)
                start_gather(j + LA, bn)

        for b in range(NBUF):
            wait_out(b)


@jax.jit
def _sc_gather(x, re):
    idx = x.astype(jnp.int32).reshape(NUM_WORKERS, N_PASSES, P_CHUNKS, CHUNK)
    re_pad = jnp.pad(re, ((0, MAX_LEN_PAD - re.shape[0]), (0, 0)))
    run = pl.kernel(
        _sc_gather_body,
        out_type=jax.ShapeDtypeStruct((N_TOKENS, D_MODEL), jnp.float32),
        mesh=plsc.VectorSubcoreMesh(core_axis_name="c", subcore_axis_name="s"),
        scratch_types=(
            [pltpu.VMEM_SHARED((MAX_LEN_PAD, D_MODEL), jnp.float32),
             pltpu.VMEM((P_CHUNKS, CHUNK), jnp.int32),
             pltpu.VMEM((NBUF, CHUNK, D_MODEL), jnp.float32)]
            + [pltpu.SemaphoreType.DMA] * (2 * NBUF)
        ),
    )
    return run(re_pad, idx)


def kernel(x, re):
    return _sc_gather(x, re)


# speedup vs baseline: 1.2271x; 1.0408x over previous
import functools

import jax
import jax.numpy as jnp
from jax import lax
from jax.experimental import pallas as pl
from jax.experimental.pallas import tpu as pltpu
from jax.experimental.pallas import tpu_sc as plsc

D_MODEL = 128
N_TOKENS = 819200
NUM_CORES = 2
NUM_SUBCORES = 16
NUM_WORKERS = NUM_CORES * NUM_SUBCORES
PER_WORKER = N_TOKENS // NUM_WORKERS
CHUNK = 80
N_CHUNKS = PER_WORKER // CHUNK
N_PASSES = 4
P_CHUNKS = N_CHUNKS // N_PASSES
NBUF = 4
LA = 2
MAX_LEN_PAD = 9088
STAGE_ROWS = MAX_LEN_PAD // NUM_SUBCORES
CHUNK_BYTES = CHUNK * D_MODEL * 4


def _sc_gather_body(table_hbm, idx_hbm, out_hbm, tbl_s, idx_v, rows_v, *sems):
    gsems, osems = sems[:NBUF], sems[NBUF:]
    sid = lax.axis_index("s")
    wid = sid * NUM_CORES + lax.axis_index("c")
    base = wid * PER_WORKER

    pltpu.sync_copy(table_hbm.at[pl.ds(sid * STAGE_ROWS, STAGE_ROWS)],
                    tbl_s.at[pl.ds(sid * STAGE_ROWS, STAGE_ROWS)])
    plsc.subcore_barrier()

    def start_gather(j, b):
        pltpu.async_copy(tbl_s.at[idx_v.at[j]], rows_v.at[b], gsems[b])

    def start_out(g, b):
        pltpu.async_copy(rows_v.at[b],
                         out_hbm.at[pl.ds(base + g * CHUNK, CHUNK)], osems[b])

    def wait_gather(b):
        pltpu.make_async_copy(table_hbm.at[pl.ds(0, CHUNK)], rows_v.at[b],
                              gsems[b]).wait()

    def wait_out(b):
        pltpu.make_async_copy(rows_v.at[b], out_hbm.at[pl.ds(0, CHUNK)],
                              osems[b]).wait()

    for p in range(N_PASSES):
        pltpu.sync_copy(idx_hbm.at[wid].at[p], idx_v)
        goff = p * P_CHUNKS
        for j in range(LA):
            start_gather(j, j)

        for b in range(NBUF):
            wait_gather(b)
            start_out(goff + b, b)
            bn = (b + LA) % NBUF
            if b + LA >= NBUF:
                wait_out(bn)
            start_gather(b + LA, bn)

        @pl.loop(NBUF, P_CHUNKS - NBUF, step=NBUF)
        def _body(jb):
            for b in range(NBUF):
                j = jb + b
                wait_gather(b)
                bn = (b + LA) % NBUF
                wait_out(bn)
                start_gather(j + LA, bn)
                start_out(goff + j, b)

        for b in range(NBUF):
            j = P_CHUNKS - NBUF + b
            wait_gather(b)
            start_out(goff + j, b)
            if j + LA < P_CHUNKS:
                bn = (b + LA) % NBUF
                wait_out(bn)
                start_gather(j + LA, bn)

        for b in range(NBUF):
            wait_out(b)


@jax.jit
def _sc_gather(x, re):
    idx = x.astype(jnp.int32).reshape(NUM_WORKERS, N_PASSES, P_CHUNKS, CHUNK)
    re_pad = jnp.pad(re, ((0, MAX_LEN_PAD - re.shape[0]), (0, 0)))
    run = pl.kernel(
        _sc_gather_body,
        out_type=jax.ShapeDtypeStruct((N_TOKENS, D_MODEL), jnp.float32),
        mesh=plsc.VectorSubcoreMesh(core_axis_name="c", subcore_axis_name="s"),
        scratch_types=(
            [pltpu.VMEM_SHARED((MAX_LEN_PAD, D_MODEL), jnp.float32),
             pltpu.VMEM((P_CHUNKS, CHUNK), jnp.int32),
             pltpu.VMEM((NBUF, CHUNK, D_MODEL), jnp.float32)]
            + [pltpu.SemaphoreType.DMA] * (2 * NBUF)
        ),
    )
    return run(re_pad, idx)


def kernel(x, re):
    return _sc_gather(x, re)


# speedup vs baseline: 1.2306x; 1.0029x over previous
import jax
import jax.numpy as jnp
from jax import lax
from jax.experimental import pallas as pl
from jax.experimental.pallas import tpu as pltpu
from jax.experimental.pallas import tpu_sc as plsc

D_MODEL = 128
N_TOKENS = 819200
NUM_CORES = 2
NUM_SUBCORES = 16
NUM_WORKERS = NUM_CORES * NUM_SUBCORES
PER_WORKER = N_TOKENS // NUM_WORKERS
CHUNK = 80
N_CHUNKS = PER_WORKER // CHUNK
N_PASSES = 4
P_CHUNKS = N_CHUNKS // N_PASSES
NBUF = 4
LA = 2
MAX_LEN_PAD = 9088
STAGE_ROWS = MAX_LEN_PAD // NUM_SUBCORES
CHUNK_BYTES = CHUNK * D_MODEL * 4


def _sc_gather_body(table_hbm, idx_hbm, out_hbm, tbl_s, idx_v, rows_v, *sems):
    gsems, osems = sems[:NBUF], sems[NBUF:]
    sid = lax.axis_index("s")
    wid = sid * NUM_CORES + lax.axis_index("c")
    base = wid * PER_WORKER

    pltpu.sync_copy(table_hbm.at[pl.ds(sid * STAGE_ROWS, STAGE_ROWS)],
                    tbl_s.at[pl.ds(sid * STAGE_ROWS, STAGE_ROWS)])
    plsc.subcore_barrier()

    def start_gather(j, b):
        pltpu.async_copy(tbl_s.at[idx_v.at[j]], rows_v.at[b], gsems[b])

    def start_out(g, b):
        pltpu.async_copy(rows_v.at[b],
                         out_hbm.at[pl.ds(base + g * CHUNK, CHUNK)], osems[b])

    def wait_gather(b):
        pltpu.make_async_copy(table_hbm.at[pl.ds(0, CHUNK)], rows_v.at[b],
                              gsems[b]).wait()

    def wait_out(b):
        pltpu.make_async_copy(rows_v.at[b], out_hbm.at[pl.ds(0, CHUNK)],
                              osems[b]).wait()

    for p in range(N_PASSES):
        pltpu.sync_copy(idx_hbm.at[wid].at[p], idx_v)
        goff = p * P_CHUNKS
        for j in range(LA):
            start_gather(j, j)

        for b in range(NBUF):
            wait_gather(b)
            start_out(goff + b, b)
            bn = (b + LA) % NBUF
            if b + LA >= NBUF:
                wait_out(bn)
            start_gather(b + LA, bn)

        @pl.loop(NBUF, P_CHUNKS - NBUF, step=NBUF)
        def _body(jb):
            for b in range(NBUF):
                j = jb + b
                wait_gather(b)
                bn = (b + LA) % NBUF
                wait_out(bn)
                start_gather(j + LA, bn)
                start_out(goff + j, b)

        for b in range(NBUF):
            j = P_CHUNKS - NBUF + b
            wait_gather(b)
            start_out(goff + j, b)
            if j + LA < P_CHUNKS:
                bn = (b + LA) % NBUF
                wait_out(bn)
                start_gather(j + LA, bn)

        for b in range(NBUF):
            wait_out(b)


@jax.jit
def _sc_gather(x, re):
    idx = x.astype(jnp.int32).reshape(NUM_WORKERS, N_PASSES, P_CHUNKS, CHUNK)
    re_pad = jnp.pad(re, ((0, MAX_LEN_PAD - re.shape[0]), (0, 0)))
    run = pl.kernel(
        _sc_gather_body,
        out_type=jax.ShapeDtypeStruct((N_TOKENS, D_MODEL), jnp.float32),
        mesh=plsc.VectorSubcoreMesh(core_axis_name="c", subcore_axis_name="s"),
        scratch_types=(
            [pltpu.VMEM_SHARED((MAX_LEN_PAD, D_MODEL), jnp.float32),
             pltpu.VMEM((P_CHUNKS, CHUNK), jnp.int32),
             pltpu.VMEM((NBUF, CHUNK, D_MODEL), jnp.float32)]
            + [pltpu.SemaphoreType.DMA] * (2 * NBUF)
        ),
    )
    return run(re_pad, idx)


def kernel(x, re):
    return _sc_gather(x, re)
